# baseline (speedup 1.0000x reference)
import functools

import jax
import jax.numpy as jnp
from jax import lax
from jax.experimental import pallas as pl
from jax.experimental.pallas import tpu as pltpu
from jax.experimental.pallas import tpu_sc as plsc

N = 10000
DH = 16
NC = 2
NS = 16
NW = NC * NS
CHUNK = 128
CH_PER_TILE = 84
E_PAD = NW * CH_PER_TILE * CHUNK
ACC_ROWS = 10112


DEG_W = 8


def _deg_body(dst3, zeros, ones_blk, out, dst_v, ones_v, acc, sem):
    c = lax.axis_index("c")
    s = lax.axis_index("s")
    wid = c * NS + s
    pltpu.sync_copy(dst3.at[wid], dst_v)
    pltpu.sync_copy(ones_blk, ones_v)
    rps = ACC_ROWS // NS
    pltpu.sync_copy(zeros.at[pl.ds(s * rps, rps)], acc.at[pl.ds(s * rps, rps)])
    plsc.subcore_barrier()

    def chunk(j, carry):
        pltpu.async_copy(ones_v, acc.at[dst_v.at[j]], sem, add=True)

        @pl.when(j >= DEG_W)
        def _():
            pltpu.make_async_copy(ones_v, acc.at[dst_v.at[j]], sem).wait()

        return carry

    lax.fori_loop(0, CH_PER_TILE, chunk, 0)

    def drain(j, carry):
        pltpu.make_async_copy(ones_v, acc.at[dst_v.at[j]], sem).wait()
        return carry

    lax.fori_loop(0, DEG_W, drain, 0)
    plsc.subcore_barrier()
    pltpu.sync_copy(acc.at[pl.ds(s * rps, rps)],
                    out.at[pl.ds(c * ACC_ROWS + s * rps, rps)])


RING = 6
PF = 3


def _agg_stream_loop(gtab, src_v, dst_v, rows_v, acc, gsems, ssems):
    for b in range(PF):
        pltpu.async_copy(gtab.at[src_v.at[b]], rows_v.at[b], gsems[b])

    def outer(t, carry):
        for k in range(RING):
            jj = RING * t + k
            pltpu.make_async_copy(gtab.at[src_v.at[k]], rows_v.at[k],
                                  gsems[k]).wait()
            pltpu.async_copy(rows_v.at[k], acc.at[dst_v.at[jj]], ssems[k],
                             add=True)
            bf = (k + PF) % RING
            nxt = jj + PF

            @pl.when(jj >= PF)
            def _():
                pltpu.make_async_copy(rows_v.at[bf], acc.at[dst_v.at[jj]],
                                      ssems[bf]).wait()

            @pl.when(nxt < CH_PER_TILE)
            def _():
                pltpu.async_copy(gtab.at[src_v.at[nxt]], rows_v.at[bf],
                                 gsems[bf])
        return carry

    lax.fori_loop(0, CH_PER_TILE // RING, outer, 0)
    for k in range(PF):
        b = (CH_PER_TILE - PF + k) % RING
        pltpu.make_async_copy(rows_v.at[b], acc.at[dst_v.at[0]],
                              ssems[b]).wait()


def _agg_body(table, src3, dst3, zeros, out,
              src_v, dst_v, rows_v, acc, gtab,
              g0, g1, g2, g3, g4, g5, s0, s1, s2, s3, s4, s5):
    c = lax.axis_index("c")
    s = lax.axis_index("s")
    wid = c * NS + s
    rps = ACC_ROWS // NS
    base = s * rps
    pltpu.sync_copy(src3.at[wid], src_v)
    pltpu.sync_copy(dst3.at[wid], dst_v)
    pltpu.sync_copy(table.at[pl.ds(base, rps)], gtab.at[pl.ds(base, rps)])
    pltpu.sync_copy(zeros.at[pl.ds(base, rps)], acc.at[pl.ds(base, rps)])
    plsc.subcore_barrier()
    _agg_stream_loop(gtab, src_v, dst_v, rows_v, acc,
                     (g0, g1, g2, g3, g4, g5), (s0, s1, s2, s3, s4, s5))
    plsc.subcore_barrier()
    pltpu.sync_copy(acc.at[pl.ds(base, rps)],
                    out.at[pl.ds(c * ACC_ROWS + base, rps)])


def _agg2_body(accp, dinv16, b1, src3, dst3, zeros, out,
               src_v, dst_v, rows_v, p0_t, p1_t, d_t, g_t, b1_t,
               acc, gtab,
               g0, g1, g2, g3, g4, g5, s0, s1, s2, s3, s4, s5):
    c = lax.axis_index("c")
    s = lax.axis_index("s")
    wid = c * NS + s
    rps = ACC_ROWS // NS
    base = s * rps
    pltpu.sync_copy(src3.at[wid], src_v)
    pltpu.sync_copy(dst3.at[wid], dst_v)
    pltpu.sync_copy(accp.at[pl.ds(base, rps)], p0_t)
    pltpu.sync_copy(accp.at[pl.ds(ACC_ROWS + base, rps)], p1_t)
    pltpu.sync_copy(dinv16.at[pl.ds(base, rps)], d_t)
    pltpu.sync_copy(b1, b1_t)
    pltpu.sync_copy(zeros.at[pl.ds(base, rps)], acc.at[pl.ds(base, rps)])

    b1v = b1_t[0]

    def row(i, carry):
        a = p0_t[i] + p1_t[i]
        d = d_t[i]
        g_t[i] = d * jnp.maximum(d * a + b1v, 0.0)
        return carry

    lax.fori_loop(0, rps, row, 0)
    pltpu.sync_copy(g_t, gtab.at[pl.ds(base, rps)])
    plsc.subcore_barrier()
    _agg_stream_loop(gtab, src_v, dst_v, rows_v, acc,
                     (g0, g1, g2, g3, g4, g5), (s0, s1, s2, s3, s4, s5))
    plsc.subcore_barrier()
    pltpu.sync_copy(acc.at[pl.ds(base, rps)],
                    out.at[pl.ds(c * ACC_ROWS + base, rps)])


_MESH = plsc.VectorSubcoreMesh(core_axis_name="c", subcore_axis_name="s")
_SC_PARAMS = pltpu.CompilerParams(use_tc_tiling_on_sc=False)

_deg_call = pl.kernel(
    _deg_body,
    out_type=jax.ShapeDtypeStruct((NC * ACC_ROWS, DH), jnp.float32),
    mesh=_MESH,
    compiler_params=_SC_PARAMS,
    scratch_types=[
        pltpu.VMEM((CH_PER_TILE, CHUNK), jnp.int32),
        pltpu.VMEM((CHUNK, DH), jnp.float32),
        pltpu.VMEM_SHARED((ACC_ROWS, DH), jnp.float32),
        pltpu.SemaphoreType.DMA,
    ],
)

_agg_call = pl.kernel(
    _agg_body,
    out_type=jax.ShapeDtypeStruct((NC * ACC_ROWS, DH), jnp.float32),
    mesh=_MESH,
    compiler_params=_SC_PARAMS,
    scratch_types=[
        pltpu.VMEM((CH_PER_TILE, CHUNK), jnp.int32),
        pltpu.VMEM((CH_PER_TILE, CHUNK), jnp.int32),
        pltpu.VMEM((RING, CHUNK, DH), jnp.float32),
        pltpu.VMEM_SHARED((ACC_ROWS, DH), jnp.float32),
        pltpu.VMEM_SHARED((ACC_ROWS, DH), jnp.float32),
    ] + [pltpu.SemaphoreType.DMA] * (2 * RING),
)

_RPS = ACC_ROWS // NS

_agg2_call = pl.kernel(
    _agg2_body,
    out_type=jax.ShapeDtypeStruct((NC * ACC_ROWS, DH), jnp.float32),
    mesh=_MESH,
    compiler_params=_SC_PARAMS,
    scratch_types=[
        pltpu.VMEM((CH_PER_TILE, CHUNK), jnp.int32),
        pltpu.VMEM((CH_PER_TILE, CHUNK), jnp.int32),
        pltpu.VMEM((RING, CHUNK, DH), jnp.float32),
        pltpu.VMEM((_RPS, DH), jnp.float32),
        pltpu.VMEM((_RPS, DH), jnp.float32),
        pltpu.VMEM((_RPS, DH), jnp.float32),
        pltpu.VMEM((_RPS, DH), jnp.float32),
        pltpu.VMEM((1, DH), jnp.float32),
        pltpu.VMEM_SHARED((ACC_ROWS, DH), jnp.float32),
        pltpu.VMEM_SHARED((ACC_ROWS, DH), jnp.float32),
    ] + [pltpu.SemaphoreType.DMA] * (2 * RING),
)


def _tc1_body(x_ref, w1_ref, degp_ref, h1p_ref, dinv_ref, dinv16_ref):
    deg = degp_ref[0:N, 0:1] + degp_ref[ACC_ROWS:ACC_ROWS + N, 0:1]
    dinv = jnp.where(deg > 0, lax.rsqrt(deg), 0.0)
    h = jnp.dot(x_ref[...], w1_ref[...], preferred_element_type=jnp.float32)
    h1p_ref[0:N, :] = h * dinv
    h1p_ref[N:ACC_ROWS, :] = jnp.zeros((ACC_ROWS - N, DH), jnp.float32)
    dinv_ref[...] = dinv
    dinv16_ref[0:N, :] = jnp.broadcast_to(dinv, (N, DH))
    dinv16_ref[N:ACC_ROWS, :] = jnp.zeros((ACC_ROWS - N, DH), jnp.float32)


def _tc3_body(accp_ref, dinv_ref, w2_ref, b2_ref, out_ref):
    a = (accp_ref[0:N, :] + accp_ref[ACC_ROWS:ACC_ROWS + N, :]) * dinv_ref[...]
    t = jnp.dot(a, w2_ref[...], preferred_element_type=jnp.float32) + b2_ref[...]
    m = jnp.max(t, axis=1, keepdims=True)
    out_ref[...] = (t - m) - jnp.log(
        jnp.sum(jnp.exp(t - m), axis=1, keepdims=True))


_tc1 = pl.pallas_call(
    _tc1_body,
    out_shape=[jax.ShapeDtypeStruct((ACC_ROWS, DH), jnp.float32),
               jax.ShapeDtypeStruct((N, 1), jnp.float32),
               jax.ShapeDtypeStruct((ACC_ROWS, DH), jnp.float32)],
)

_tc3 = pl.pallas_call(
    _tc3_body,
    out_shape=jax.ShapeDtypeStruct((N, 2), jnp.float32),
)


def kernel(x, edge_index, W1, b1, W2, b2):
    e = edge_index.astype(jnp.int32)
    loops = jnp.arange(N, dtype=jnp.int32)
    src = jnp.concatenate([e[0], loops])
    dst = jnp.concatenate([e[1], loops])
    pad = E_PAD - src.shape[0]
    src = jnp.concatenate([src, jnp.zeros((pad,), jnp.int32)])
    dst = jnp.concatenate([dst, jnp.full((pad,), N, jnp.int32)])
    src3 = src.reshape(NW, CH_PER_TILE, CHUNK)
    dst3 = dst.reshape(NW, CH_PER_TILE, CHUNK)

    zeros = jnp.zeros((ACC_ROWS, DH), jnp.float32)
    ones_blk = jnp.ones((CHUNK, DH), jnp.float32)

    degp = _deg_call(dst3, zeros, ones_blk)
    h1p, dinv, dinv16 = _tc1(x, W1, degp)
    acc1 = _agg_call(h1p, src3, dst3, zeros)
    acc2 = _agg2_call(acc1, dinv16, b1.reshape(1, DH), src3, dst3, zeros)
    return _tc3(acc2, dinv, W2, b2.reshape(1, 2))


# speedup vs baseline: 1.1588x; 1.1588x over previous
import functools

import jax
import jax.numpy as jnp
from jax import lax
from jax.experimental import pallas as pl
from jax.experimental.pallas import tpu as pltpu
from jax.experimental.pallas import tpu_sc as plsc

N = 10000
DH = 16
NC = 2
NS = 16
NW = NC * NS
CHUNK = 128
CH_PER_TILE = 81
E_PAD = NW * CH_PER_TILE * CHUNK
ACC_ROWS = 10112


DEG_W = 8


def _deg_body(dst3, zeros, ones_blk, out, dst_v, ones_v, acc, sem):
    c = lax.axis_index("c")
    s = lax.axis_index("s")
    wid = c * NS + s
    pltpu.sync_copy(dst3.at[wid], dst_v)
    pltpu.sync_copy(ones_blk, ones_v)
    rps = ACC_ROWS // NS
    pltpu.sync_copy(zeros.at[pl.ds(s * rps, rps)], acc.at[pl.ds(s * rps, rps)])
    plsc.subcore_barrier()

    def chunk(j, carry):
        pltpu.async_copy(ones_v, acc.at[dst_v.at[j]], sem, add=True)

        @pl.when(j >= DEG_W)
        def _():
            pltpu.make_async_copy(ones_v, acc.at[dst_v.at[j]], sem).wait()

        return carry

    lax.fori_loop(0, CH_PER_TILE, chunk, 0)

    def drain(j, carry):
        pltpu.make_async_copy(ones_v, acc.at[dst_v.at[j]], sem).wait()
        return carry

    lax.fori_loop(0, DEG_W, drain, 0)
    plsc.subcore_barrier()
    pltpu.sync_copy(acc.at[pl.ds(s * rps, rps)],
                    out.at[pl.ds(c * ACC_ROWS + s * rps, rps)])


RING = 3


def _agg_stream_loop(gtab, src_v, dst_v, rows_v, acc, gsems):
    for b in range(RING):
        pltpu.async_copy(gtab.at[src_v.at[b]], rows_v.at[b], gsems[b])

    def outer(t, carry):
        for b in range(RING):
            jj = RING * t + b
            pltpu.make_async_copy(gtab.at[src_v.at[b]], rows_v.at[b],
                                  gsems[b]).wait()
            pltpu.sync_copy(rows_v.at[b], acc.at[dst_v.at[jj]], add=True)
            nxt = jj + RING

            @pl.when(nxt < CH_PER_TILE)
            def _():
                pltpu.async_copy(gtab.at[src_v.at[nxt]], rows_v.at[b],
                                 gsems[b])
        return carry

    lax.fori_loop(0, CH_PER_TILE // RING, outer, 0)


def _agg_body(table, src3, dst3, zeros, out,
              src_v, dst_v, rows_v, acc, gtab, g0, g1, g2):
    c = lax.axis_index("c")
    s = lax.axis_index("s")
    wid = c * NS + s
    rps = ACC_ROWS // NS
    base = s * rps
    pltpu.sync_copy(src3.at[wid], src_v)
    pltpu.sync_copy(dst3.at[wid], dst_v)
    pltpu.sync_copy(table.at[pl.ds(base, rps)], gtab.at[pl.ds(base, rps)])
    pltpu.sync_copy(zeros.at[pl.ds(base, rps)], acc.at[pl.ds(base, rps)])
    plsc.subcore_barrier()
    _agg_stream_loop(gtab, src_v, dst_v, rows_v, acc, (g0, g1, g2))
    plsc.subcore_barrier()
    pltpu.sync_copy(acc.at[pl.ds(base, rps)],
                    out.at[pl.ds(c * ACC_ROWS + base, rps)])


def _agg2_body(accp, dinv16, b1, src3, dst3, zeros, out,
               src_v, dst_v, rows_v, p0_t, p1_t, d_t, g_t, b1_t,
               acc, gtab, g0, g1, g2):
    c = lax.axis_index("c")
    s = lax.axis_index("s")
    wid = c * NS + s
    rps = ACC_ROWS // NS
    base = s * rps
    pltpu.sync_copy(src3.at[wid], src_v)
    pltpu.sync_copy(dst3.at[wid], dst_v)
    pltpu.sync_copy(accp.at[pl.ds(base, rps)], p0_t)
    pltpu.sync_copy(accp.at[pl.ds(ACC_ROWS + base, rps)], p1_t)
    pltpu.sync_copy(dinv16.at[pl.ds(base, rps)], d_t)
    pltpu.sync_copy(b1, b1_t)
    pltpu.sync_copy(zeros.at[pl.ds(base, rps)], acc.at[pl.ds(base, rps)])

    b1v = b1_t[0]

    def row(i, carry):
        a = p0_t[i] + p1_t[i]
        d = d_t[i]
        g_t[i] = d * jnp.maximum(d * a + b1v, 0.0)
        return carry

    lax.fori_loop(0, rps, row, 0)
    pltpu.sync_copy(g_t, gtab.at[pl.ds(base, rps)])
    plsc.subcore_barrier()
    _agg_stream_loop(gtab, src_v, dst_v, rows_v, acc, (g0, g1, g2))
    plsc.subcore_barrier()
    pltpu.sync_copy(acc.at[pl.ds(base, rps)],
                    out.at[pl.ds(c * ACC_ROWS + base, rps)])


_MESH = plsc.VectorSubcoreMesh(core_axis_name="c", subcore_axis_name="s")
_SC_PARAMS = pltpu.CompilerParams(use_tc_tiling_on_sc=False)

_deg_call = pl.kernel(
    _deg_body,
    out_type=jax.ShapeDtypeStruct((NC * ACC_ROWS, DH), jnp.float32),
    mesh=_MESH,
    compiler_params=_SC_PARAMS,
    scratch_types=[
        pltpu.VMEM((CH_PER_TILE, CHUNK), jnp.int32),
        pltpu.VMEM((CHUNK, DH), jnp.float32),
        pltpu.VMEM_SHARED((ACC_ROWS, DH), jnp.float32),
        pltpu.SemaphoreType.DMA,
    ],
)

_agg_call = pl.kernel(
    _agg_body,
    out_type=jax.ShapeDtypeStruct((NC * ACC_ROWS, DH), jnp.float32),
    mesh=_MESH,
    compiler_params=_SC_PARAMS,
    scratch_types=[
        pltpu.VMEM((CH_PER_TILE, CHUNK), jnp.int32),
        pltpu.VMEM((CH_PER_TILE, CHUNK), jnp.int32),
        pltpu.VMEM((RING, CHUNK, DH), jnp.float32),
        pltpu.VMEM_SHARED((ACC_ROWS, DH), jnp.float32),
        pltpu.VMEM_SHARED((ACC_ROWS, DH), jnp.float32),
    ] + [pltpu.SemaphoreType.DMA] * RING,
)

_RPS = ACC_ROWS // NS

_agg2_call = pl.kernel(
    _agg2_body,
    out_type=jax.ShapeDtypeStruct((NC * ACC_ROWS, DH), jnp.float32),
    mesh=_MESH,
    compiler_params=_SC_PARAMS,
    scratch_types=[
        pltpu.VMEM((CH_PER_TILE, CHUNK), jnp.int32),
        pltpu.VMEM((CH_PER_TILE, CHUNK), jnp.int32),
        pltpu.VMEM((RING, CHUNK, DH), jnp.float32),
        pltpu.VMEM((_RPS, DH), jnp.float32),
        pltpu.VMEM((_RPS, DH), jnp.float32),
        pltpu.VMEM((_RPS, DH), jnp.float32),
        pltpu.VMEM((_RPS, DH), jnp.float32),
        pltpu.VMEM((1, DH), jnp.float32),
        pltpu.VMEM_SHARED((ACC_ROWS, DH), jnp.float32),
        pltpu.VMEM_SHARED((ACC_ROWS, DH), jnp.float32),
    ] + [pltpu.SemaphoreType.DMA] * RING,
)


def _tc1_body(x_ref, w1_ref, degp_ref, h1p_ref, dinv_ref, dinv16_ref):
    deg = degp_ref[0:N, 0:1] + degp_ref[ACC_ROWS:ACC_ROWS + N, 0:1]
    dinv = jnp.where(deg > 0, lax.rsqrt(deg), 0.0)
    h = jnp.dot(x_ref[...], w1_ref[...], preferred_element_type=jnp.float32)
    h1p_ref[0:N, :] = h * dinv
    h1p_ref[N:ACC_ROWS, :] = jnp.zeros((ACC_ROWS - N, DH), jnp.float32)
    dinv_ref[...] = dinv
    dinv16_ref[0:N, :] = jnp.broadcast_to(dinv, (N, DH))
    dinv16_ref[N:ACC_ROWS, :] = jnp.zeros((ACC_ROWS - N, DH), jnp.float32)


def _tc3_body(accp_ref, dinv_ref, w2_ref, b2_ref, out_ref):
    a = (accp_ref[0:N, :] + accp_ref[ACC_ROWS:ACC_ROWS + N, :]) * dinv_ref[...]
    t = jnp.dot(a, w2_ref[...], preferred_element_type=jnp.float32) + b2_ref[...]
    m = jnp.max(t, axis=1, keepdims=True)
    out_ref[...] = (t - m) - jnp.log(
        jnp.sum(jnp.exp(t - m), axis=1, keepdims=True))


_tc1 = pl.pallas_call(
    _tc1_body,
    out_shape=[jax.ShapeDtypeStruct((ACC_ROWS, DH), jnp.float32),
               jax.ShapeDtypeStruct((N, 1), jnp.float32),
               jax.ShapeDtypeStruct((ACC_ROWS, DH), jnp.float32)],
)

_tc3 = pl.pallas_call(
    _tc3_body,
    out_shape=jax.ShapeDtypeStruct((N, 2), jnp.float32),
)


def kernel(x, edge_index, W1, b1, W2, b2):
    e = edge_index.astype(jnp.int32)
    loops = jnp.arange(N, dtype=jnp.int32)
    src = jnp.concatenate([e[0], loops])
    dst = jnp.concatenate([e[1], loops])
    pad = E_PAD - src.shape[0]
    src = jnp.concatenate([src, jnp.zeros((pad,), jnp.int32)])
    dst = jnp.concatenate([dst, jnp.full((pad,), N, jnp.int32)])
    src3 = src.reshape(NW, CH_PER_TILE, CHUNK)
    dst3 = dst.reshape(NW, CH_PER_TILE, CHUNK)

    zeros = jnp.zeros((ACC_ROWS, DH), jnp.float32)
    ones_blk = jnp.ones((CHUNK, DH), jnp.float32)

    degp = _deg_call(dst3, zeros, ones_blk)
    h1p, dinv, dinv16 = _tc1(x, W1, degp)
    acc1 = _agg_call(h1p, src3, dst3, zeros)
    acc2 = _agg2_call(acc1, dinv16, b1.reshape(1, DH), src3, dst3, zeros)
    return _tc3(acc2, dinv, W2, b2.reshape(1, 2))


# speedup vs baseline: 1.1598x; 1.0009x over previous
import functools

import jax
import jax.numpy as jnp
from jax import lax
from jax.experimental import pallas as pl
from jax.experimental.pallas import tpu as pltpu
from jax.experimental.pallas import tpu_sc as plsc

N = 10000
DH = 16
NC = 2
NS = 16
NW = NC * NS
CHUNK = 128
CH_PER_TILE = 81
E_PAD = NW * CH_PER_TILE * CHUNK
ACC_ROWS = 10112


DEG_W = 8


def _deg_body(dst3, zeros, ones_blk, out, dst_v, ones_v, acc, sem):
    c = lax.axis_index("c")
    s = lax.axis_index("s")
    wid = c * NS + s
    pltpu.sync_copy(dst3.at[wid], dst_v)
    pltpu.sync_copy(ones_blk, ones_v)
    rps = ACC_ROWS // NS
    pltpu.sync_copy(zeros.at[pl.ds(s * rps, rps)], acc.at[pl.ds(s * rps, rps)])
    plsc.subcore_barrier()

    def chunk(j, carry):
        pltpu.async_copy(ones_v, acc.at[dst_v.at[j]], sem, add=True)

        @pl.when(j >= DEG_W)
        def _():
            pltpu.make_async_copy(ones_v, acc.at[dst_v.at[j]], sem).wait()

        return carry

    lax.fori_loop(0, CH_PER_TILE, chunk, 0)

    def drain(j, carry):
        pltpu.make_async_copy(ones_v, acc.at[dst_v.at[j]], sem).wait()
        return carry

    lax.fori_loop(0, DEG_W, drain, 0)
    plsc.subcore_barrier()
    pltpu.sync_copy(acc.at[pl.ds(s * rps, rps)],
                    out.at[pl.ds(c * ACC_ROWS + s * rps, rps)])


RING = 3


def _agg_stream_loop(gtab, src_v, dst_v, rows_v, acc, gsems):
    for b in range(RING):
        pltpu.async_copy(gtab.at[src_v.at[b]], rows_v.at[b], gsems[b])

    def outer(t, carry):
        for b in range(RING):
            jj = RING * t + b
            pltpu.make_async_copy(gtab.at[src_v.at[b]], rows_v.at[b],
                                  gsems[b]).wait()
            pltpu.sync_copy(rows_v.at[b], acc.at[dst_v.at[jj]], add=True)
            nxt = jj + RING

            @pl.when(nxt < CH_PER_TILE)
            def _():
                pltpu.async_copy(gtab.at[src_v.at[nxt]], rows_v.at[b],
                                 gsems[b])
        return carry

    lax.fori_loop(0, CH_PER_TILE // RING, outer, 0)


def _agg1_body(h, degp, src3, dst3, zeros, out, d16_out,
               src_v, dst_v, rows_v, h_t, p0_t, p1_t, acc, gtab, g0, g1, g2):
    c = lax.axis_index("c")
    s = lax.axis_index("s")
    wid = c * NS + s
    rps = ACC_ROWS // NS
    base = s * rps
    pltpu.sync_copy(src3.at[wid], src_v)
    pltpu.sync_copy(dst3.at[wid], dst_v)
    pltpu.sync_copy(h.at[pl.ds(base, rps)], h_t)
    pltpu.sync_copy(degp.at[pl.ds(base, rps)], p0_t)
    pltpu.sync_copy(degp.at[pl.ds(ACC_ROWS + base, rps)], p1_t)
    pltpu.sync_copy(zeros.at[pl.ds(base, rps)], acc.at[pl.ds(base, rps)])

    def row(i, carry):
        a = p0_t[i] + p1_t[i]
        bits = lax.bitcast_convert_type(a, jnp.int32)
        seed = jnp.int32(0x5F3759DF) - lax.shift_right_logical(bits, 1)
        d = lax.bitcast_convert_type(seed, jnp.float32)
        half = -0.5 * a
        for _ in range(3):
            d = d * (1.5 + half * d * d)
        p0_t[i] = d
        h_t[i] = d * h_t[i]
        return carry

    lax.fori_loop(0, rps, row, 0)
    pltpu.sync_copy(h_t, gtab.at[pl.ds(base, rps)])
    pltpu.sync_copy(p0_t, d16_out.at[pl.ds(c * ACC_ROWS + base, rps)])
    plsc.subcore_barrier()
    _agg_stream_loop(gtab, src_v, dst_v, rows_v, acc, (g0, g1, g2))
    plsc.subcore_barrier()
    pltpu.sync_copy(acc.at[pl.ds(base, rps)],
                    out.at[pl.ds(c * ACC_ROWS + base, rps)])


def _agg2_body(accp, dinv16, b1, src3, dst3, zeros, out,
               src_v, dst_v, rows_v, p0_t, p1_t, d_t, g_t, b1_t,
               acc, gtab, g0, g1, g2):
    c = lax.axis_index("c")
    s = lax.axis_index("s")
    wid = c * NS + s
    rps = ACC_ROWS // NS
    base = s * rps
    pltpu.sync_copy(src3.at[wid], src_v)
    pltpu.sync_copy(dst3.at[wid], dst_v)
    pltpu.sync_copy(accp.at[pl.ds(base, rps)], p0_t)
    pltpu.sync_copy(accp.at[pl.ds(ACC_ROWS + base, rps)], p1_t)
    pltpu.sync_copy(dinv16.at[pl.ds(c * ACC_ROWS + base, rps)], d_t)
    pltpu.sync_copy(b1, b1_t)
    pltpu.sync_copy(zeros.at[pl.ds(base, rps)], acc.at[pl.ds(base, rps)])

    b1v = b1_t[0]

    def row(i, carry):
        a = p0_t[i] + p1_t[i]
        d = d_t[i]
        g_t[i] = d * jnp.maximum(d * a + b1v, 0.0)
        return carry

    lax.fori_loop(0, rps, row, 0)
    pltpu.sync_copy(g_t, gtab.at[pl.ds(base, rps)])
    plsc.subcore_barrier()
    _agg_stream_loop(gtab, src_v, dst_v, rows_v, acc, (g0, g1, g2))
    plsc.subcore_barrier()
    pltpu.sync_copy(acc.at[pl.ds(base, rps)],
                    out.at[pl.ds(c * ACC_ROWS + base, rps)])


_MESH = plsc.VectorSubcoreMesh(core_axis_name="c", subcore_axis_name="s")
_SC_PARAMS = pltpu.CompilerParams(use_tc_tiling_on_sc=False)

_deg_call = pl.kernel(
    _deg_body,
    out_type=jax.ShapeDtypeStruct((NC * ACC_ROWS, DH), jnp.float32),
    mesh=_MESH,
    compiler_params=_SC_PARAMS,
    scratch_types=[
        pltpu.VMEM((CH_PER_TILE, CHUNK), jnp.int32),
        pltpu.VMEM((CHUNK, DH), jnp.float32),
        pltpu.VMEM_SHARED((ACC_ROWS, DH), jnp.float32),
        pltpu.SemaphoreType.DMA,
    ],
)

_RPS = ACC_ROWS // NS

_agg1_call = pl.kernel(
    _agg1_body,
    out_type=[jax.ShapeDtypeStruct((NC * ACC_ROWS, DH), jnp.float32),
              jax.ShapeDtypeStruct((NC * ACC_ROWS, DH), jnp.float32)],
    mesh=_MESH,
    compiler_params=_SC_PARAMS,
    scratch_types=[
        pltpu.VMEM((CH_PER_TILE, CHUNK), jnp.int32),
        pltpu.VMEM((CH_PER_TILE, CHUNK), jnp.int32),
        pltpu.VMEM((RING, CHUNK, DH), jnp.float32),
        pltpu.VMEM((_RPS, DH), jnp.float32),
        pltpu.VMEM((_RPS, DH), jnp.float32),
        pltpu.VMEM((_RPS, DH), jnp.float32),
        pltpu.VMEM_SHARED((ACC_ROWS, DH), jnp.float32),
        pltpu.VMEM_SHARED((ACC_ROWS, DH), jnp.float32),
    ] + [pltpu.SemaphoreType.DMA] * RING,
)

_agg2_call = pl.kernel(
    _agg2_body,
    out_type=jax.ShapeDtypeStruct((NC * ACC_ROWS, DH), jnp.float32),
    mesh=_MESH,
    compiler_params=_SC_PARAMS,
    scratch_types=[
        pltpu.VMEM((CH_PER_TILE, CHUNK), jnp.int32),
        pltpu.VMEM((CH_PER_TILE, CHUNK), jnp.int32),
        pltpu.VMEM((RING, CHUNK, DH), jnp.float32),
        pltpu.VMEM((_RPS, DH), jnp.float32),
        pltpu.VMEM((_RPS, DH), jnp.float32),
        pltpu.VMEM((_RPS, DH), jnp.float32),
        pltpu.VMEM((_RPS, DH), jnp.float32),
        pltpu.VMEM((1, DH), jnp.float32),
        pltpu.VMEM_SHARED((ACC_ROWS, DH), jnp.float32),
        pltpu.VMEM_SHARED((ACC_ROWS, DH), jnp.float32),
    ] + [pltpu.SemaphoreType.DMA] * RING,
)


def _mm1_body(x_ref, w1_ref, h_ref):
    h_ref[0:N, :] = jnp.dot(x_ref[...], w1_ref[...],
                            preferred_element_type=jnp.float32)
    h_ref[N:ACC_ROWS, :] = jnp.zeros((ACC_ROWS - N, DH), jnp.float32)


def _tc3_body(accp_ref, degp_ref, w2_ref, b2_ref, out_ref):
    deg = degp_ref[0:N, 0:1] + degp_ref[ACC_ROWS:ACC_ROWS + N, 0:1]
    dinv = jnp.where(deg > 0, lax.rsqrt(deg), 0.0)
    a = (accp_ref[0:N, :] + accp_ref[ACC_ROWS:ACC_ROWS + N, :]) * dinv
    t = jnp.dot(a, w2_ref[...], preferred_element_type=jnp.float32) + b2_ref[...]
    m = jnp.max(t, axis=1, keepdims=True)
    out_ref[...] = (t - m) - jnp.log(
        jnp.sum(jnp.exp(t - m), axis=1, keepdims=True))


_mm1 = pl.pallas_call(
    _mm1_body,
    out_shape=jax.ShapeDtypeStruct((ACC_ROWS, DH), jnp.float32),
)

_tc3 = pl.pallas_call(
    _tc3_body,
    out_shape=jax.ShapeDtypeStruct((N, 2), jnp.float32),
)


def kernel(x, edge_index, W1, b1, W2, b2):
    e = edge_index.astype(jnp.int32)
    loops = jnp.arange(N, dtype=jnp.int32)
    src = jnp.concatenate([e[0], loops])
    dst = jnp.concatenate([e[1], loops])
    pad = E_PAD - src.shape[0]
    src = jnp.concatenate([src, jnp.zeros((pad,), jnp.int32)])
    dst = jnp.concatenate([dst, jnp.full((pad,), N, jnp.int32)])
    src3 = src.reshape(NW, CH_PER_TILE, CHUNK)
    dst3 = dst.reshape(NW, CH_PER_TILE, CHUNK)

    zeros = jnp.zeros((ACC_ROWS, DH), jnp.float32)
    ones_blk = jnp.ones((CHUNK, DH), jnp.float32)

    h = _mm1(x, W1)
    degp = _deg_call(dst3, zeros, ones_blk)
    acc1, d16 = _agg1_call(h, degp, src3, dst3, zeros)
    acc2 = _agg2_call(acc1, d16, b1.reshape(1, DH), src3, dst3, zeros)
    return _tc3(acc2, degp, W2, b2.reshape(1, 2))
